# baseline (speedup 1.0000x reference)
import functools

import jax
import jax.numpy as jnp
from jax import lax
from jax.experimental import pallas as pl
from jax.experimental.pallas import tpu as pltpu
from jax.experimental.pallas import tpu_sc as plsc

N = 10000
IN_F = 128
OUT_F = 32
NRELA = 16
E1 = 256000
E2 = 64000
E_TOT = E1 + E2

NC = 2
NS = 16
NW = NC * NS
K = 256
NCHUNKS = E_TOT // K
C1 = E1 // K
SB = 128
NSB = K // SB
NG = K // 16
W = 48
NPAD = 10240
RPT = NPAD // NS


def _pack48(w_main, w2_ref):
    scal = lax.dot(w_main, w2_ref, precision=lax.Precision.HIGHEST)
    pad = jnp.zeros((w_main.shape[0], W - OUT_F - 1), jnp.float32)
    return jnp.concatenate([w_main, scal, pad], axis=1)


def _node_body(x_ref, wa_ref, wb_ref, w2_ref, a_ref, b_ref):
    xb = x_ref[...]
    wa48 = _pack48(wa_ref[...], w2_ref[...])
    wb48 = _pack48(wb_ref[...], w2_ref[...])
    a_ref[...] = lax.dot(xb, wa48, precision=lax.Precision.HIGHEST)
    b_ref[...] = lax.dot(xb, wb48, precision=lax.Precision.HIGHEST)


def _node_precompute(x, w1a_t, w1b_t, w2_t):
    blk = 2000
    return pl.pallas_call(
        _node_body,
        grid=(N // blk,),
        in_specs=[
            pl.BlockSpec((blk, IN_F), lambda i: (i, 0)),
            pl.BlockSpec((IN_F, OUT_F), lambda i: (0, 0)),
            pl.BlockSpec((IN_F, OUT_F), lambda i: (0, 0)),
            pl.BlockSpec((OUT_F, 1), lambda i: (0, 0)),
        ],
        out_specs=[
            pl.BlockSpec((blk, W), lambda i: (i, 0)),
            pl.BlockSpec((blk, W), lambda i: (i, 0)),
        ],
        out_shape=[
            jax.ShapeDtypeStruct((N, W), jnp.float32),
            jax.ShapeDtypeStruct((N, W), jnp.float32),
        ],
    )(x, w1a_t, w1b_t, w2_t)


def _edge_body(ee_ref, wc_ref, w2_ref, g_ref):
    w48 = _pack48(wc_ref[...], w2_ref[...])
    pieces = []
    for i in range(8):
        row = [w48]
        if i > 0:
            row.insert(0, jnp.zeros((NRELA, W * i), jnp.float32))
        if i < 7:
            row.append(jnp.zeros((NRELA, W * (7 - i)), jnp.float32))
        pieces.append(jnp.concatenate(row, axis=1))
    wbd = jnp.concatenate(pieces, axis=0)
    g_ref[...] = lax.dot(ee_ref[...], wbd, precision=lax.Precision.HIGHEST)


def _edge_precompute(ee8, w1c_t, w2_t):
    blk = 4000
    nr = ee8.shape[0]
    return pl.pallas_call(
        _edge_body,
        grid=(nr // blk,),
        in_specs=[
            pl.BlockSpec((blk, 8 * NRELA), lambda i: (i, 0)),
            pl.BlockSpec((NRELA, OUT_F), lambda i: (0, 0)),
            pl.BlockSpec((OUT_F, 1), lambda i: (0, 0)),
        ],
        out_specs=pl.BlockSpec((blk, 8 * W), lambda i: (i, 0)),
        out_shape=jax.ShapeDtypeStruct((nr, 8 * W), jnp.float32),
    )(ee8, w1c_t, w2_t)


_BCAST_DNUMS = lax.GatherDimensionNumbers(
    offset_dims=(), collapsed_slice_dims=(0,), start_index_map=(0,))

_MESH = plsc.VectorSubcoreMesh(
    core_axis_name="c", subcore_axis_name="s", num_cores=NC)


_SC_SCRATCH = []
for _b in range(2):
    _SC_SCRATCH += [
        pltpu.VMEM((NSB, SB), jnp.int32),
        pltpu.VMEM((NSB, SB), jnp.int32),
        pltpu.VMEM((NSB, SB), jnp.int32),
        pltpu.VMEM((K, W), jnp.float32),
        pltpu.VMEM((K, W), jnp.float32),
        pltpu.VMEM((K, W), jnp.float32),
        pltpu.VMEM((K, W), jnp.float32),
    ]
_SC_SCRATCH += [pltpu.VMEM_SHARED((NPAD, W), jnp.float32)]
_SC_SCRATCH += [pltpu.SemaphoreType.DMA] * 8


@functools.partial(
    pl.kernel,
    out_type=jax.ShapeDtypeStruct((NC * NPAD, W), jnp.float32),
    mesh=_MESH,
    compiler_params=pltpu.CompilerParams(
        needs_layout_passes=False, use_tc_tiling_on_sc=False),
    scratch_types=_SC_SCRATCH,
)
def _sc_aggregate(a_hbm, b_hbm, gp1_hbm, gp2_hbm,
                  src1_hbm, dst1_hbm, src2_hbm, dst2_hbm, out_hbm,
                  ig_s0, ig_d0, is_s0, ra0, rb0, gv0, wc0,
                  ig_s1, ig_d1, is_s1, ra1, rb1, gv1, wc1,
                  acc, sem_ig0, sem_ig1, sem_is0, sem_is1,
                  sem_g0, sem_g1, sem_sc0, sem_sc1):
    cid = lax.axis_index("c")
    sid = lax.axis_index("s")
    wid = cid * NS + sid
    over = jnp.maximum(wid - 17, 0)
    start = 40 * wid - 2 * over
    nch = jnp.where(wid >= 17, 38, 40)

    bufs = ((ig_s0, ig_d0, is_s0, ra0, rb0, gv0, wc0),
            (ig_s1, ig_d1, is_s1, ra1, rb1, gv1, wc1))
    sem_ig = (sem_ig0, sem_ig1)
    sem_is = (sem_is0, sem_is1)
    sem_g = (sem_g0, sem_g1)
    sem_sc = (sem_sc0, sem_sc1)

    def issue_idx(c, dst_s, dst_d, sem):
        @pl.when(c < C1)
        def _():
            pltpu.async_copy(src1_hbm.at[c], dst_s, sem)
            if dst_d is not None:
                pltpu.async_copy(dst1_hbm.at[c], dst_d, sem)

        @pl.when(c >= C1)
        def _():
            pltpu.async_copy(src2_hbm.at[c - C1], dst_s, sem)
            if dst_d is not None:
                pltpu.async_copy(dst2_hbm.at[c - C1], dst_d, sem)

    def issue_gathers(c, b, sem):
        ig_si, ig_di, _, ra, rb, gv, _ = bufs[b]

        @pl.when(c < C1)
        def _():
            pltpu.async_copy(gp1_hbm.at[pl.ds(c * K, K)], gv, sem)

        @pl.when(c >= C1)
        def _():
            pltpu.async_copy(gp2_hbm.at[pl.ds((c - C1) * K, K)], gv, sem)

        for s in range(NSB):
            pltpu.async_copy(a_hbm.at[ig_si.at[s]],
                             ra.at[pl.ds(s * SB, SB)], sem)
            pltpu.async_copy(b_hbm.at[ig_di.at[s]],
                             rb.at[pl.ds(s * SB, SB)], sem)

    def drain_gathers(b):
        _, _, _, ra, rb, gv, _ = bufs[b]
        sem = sem_g[b]
        pltpu.make_async_copy(gp1_hbm.at[pl.ds(0, K)], gv, sem).wait()
        for s in range(NSB):
            pltpu.make_async_copy(a_hbm.at[pl.ds(0, SB)],
                                  ra.at[pl.ds(s * SB, SB)], sem).wait()
            pltpu.make_async_copy(a_hbm.at[pl.ds(0, SB)],
                                  rb.at[pl.ds(s * SB, SB)], sem).wait()

    def drain_idx(b, scatter):
        if scatter:
            dst, sem = bufs[b][2], sem_is[b]
            pltpu.make_async_copy(src1_hbm.at[0], dst, sem).wait()
        else:
            sem = sem_ig[b]
            pltpu.make_async_copy(src1_hbm.at[0], bufs[b][0], sem).wait()
            pltpu.make_async_copy(src1_hbm.at[0], bufs[b][1], sem).wait()

    def issue_scatter(b):
        is_si, wc = bufs[b][2], bufs[b][6]
        for s in range(NSB):
            pltpu.async_copy(wc.at[pl.ds(s * SB, SB)],
                             acc.at[is_si.at[s]], sem_sc[b], add=True)

    def drain_scatter(b):
        wc = bufs[b][6]
        for s in range(NSB):
            pltpu.make_async_copy(gp1_hbm.at[pl.ds(0, SB)],
                                  wc.at[pl.ds(s * SB, SB)],
                                  sem_sc[b]).wait()

    def compute(b):
        _, _, _, ra, rb, gv, wc = bufs[b]

        @pl.loop(0, NG)
        def _group(g):
            gb = g * 16
            rows16 = jnp.full((16,), gb, jnp.int32) + lax.iota(jnp.int32, 16)
            col32 = jnp.full((16,), OUT_F, jnp.int32)
            a2g = plsc.load_gather(ra, [rows16, col32])
            b2g = plsc.load_gather(rb, [rows16, col32])
            g2g = plsc.load_gather(gv, [rows16, col32])
            braw = a2g + b2g + g2g
            w16 = jnp.exp(jnp.maximum(braw, 0.2 * braw))
            plsc.store_scatter(wc, [rows16, col32], w16)
            for k in range(16):
                wb = lax.gather(
                    w16, jnp.full((16, 1), k, jnp.int32),
                    _BCAST_DNUMS, (1,),
                    mode=lax.GatherScatterMode.PROMISE_IN_BOUNDS)
                r = gb + k
                c0 = (ra[r, pl.ds(0, 16)] + rb[r, pl.ds(0, 16)]
                      + gv[r, pl.ds(0, 16)])
                c1 = (ra[r, pl.ds(16, 16)] + rb[r, pl.ds(16, 16)]
                      + gv[r, pl.ds(16, 16)])
                wc[r, pl.ds(0, 16)] = wb * c0
                wc[r, pl.ds(16, 16)] = wb * c1

    zero = jnp.zeros((16,), jnp.float32)

    @pl.loop(0, K)
    def _zero_row(r):
        for c0 in range(0, W, 16):
            wc0[r, pl.ds(c0, 16)] = zero
            wc1[r, pl.ds(c0, 16)] = zero

    for t in range(RPT // 128):
        pltpu.sync_copy(wc0.at[pl.ds(0, 128)],
                        acc.at[pl.ds(sid * RPT + t * 128, 128)])
    plsc.subcore_barrier()

    @pl.when(start < C1)
    def _():
        pltpu.sync_copy(src1_hbm.at[start], ig_s0)
        pltpu.sync_copy(dst1_hbm.at[start], ig_d0)

    @pl.when(start >= C1)
    def _():
        pltpu.sync_copy(src2_hbm.at[start - C1], ig_s0)
        pltpu.sync_copy(dst2_hbm.at[start - C1], ig_d0)

    issue_gathers(start, 0, sem_g[0])
    issue_idx(start + 1, ig_s1, ig_d1, sem_ig[1])

    def part(l, b):
        o = 1 - b
        c = start + l

        @pl.when(l > 0)
        def _():
            drain_scatter(o)

        drain_gathers(b)
        issue_idx(c, bufs[b][2], None, sem_is[b])

        @pl.when(l + 2 < nch)
        def _():
            issue_idx(c + 2, bufs[b][0], bufs[b][1], sem_ig[b])

        compute(b)

        @pl.when(l + 1 < nch)
        def _():
            drain_idx(o, scatter=False)
            issue_gathers(c + 1, o, sem_g[o])

        drain_idx(b, scatter=True)
        issue_scatter(b)

    @pl.loop(0, 20)
    def _pair(p):
        @pl.when(2 * p < nch)
        def _():
            part(2 * p, 0)
            part(2 * p + 1, 1)

    drain_scatter(1)

    plsc.subcore_barrier()
    pltpu.sync_copy(acc.at[pl.ds(sid * RPT, RPT)],
                    out_hbm.at[pl.ds(cid * NPAD + sid * RPT, RPT)])


def _norm_body(p_ref, o_ref):
    s = p_ref[0] + p_ref[1]
    num = s[:, 0:OUT_F]
    den = s[:, OUT_F:OUT_F + 1]
    o_ref[...] = num / (den + 1e-12)


def _normalize(p):
    blk = 1000
    return pl.pallas_call(
        _norm_body,
        grid=(N // blk,),
        in_specs=[pl.BlockSpec((2, blk, W), lambda i: (0, i, 0))],
        out_specs=pl.BlockSpec((blk, OUT_F), lambda i: (i, 0)),
        out_shape=jax.ShapeDtypeStruct((N, OUT_F), jnp.float32),
    )(p)


def kernel(x, edges, edge_embed, nhop_edges, nhop_edge_embed, w1, w2):
    x = x.astype(jnp.float32)

    w1a_t = w1[:, :IN_F].T
    w1b_t = w1[:, IN_F:2 * IN_F].T
    w1c_t = w1[:, 2 * IN_F:].T
    w2_t = w2.T

    a_tab, b_tab = _node_precompute(x, w1a_t, w1b_t, w2_t)
    ee1 = edge_embed.astype(jnp.float32).reshape(E1 // 8, 8 * NRELA)
    ee2 = nhop_edge_embed.astype(jnp.float32).reshape(E2 // 8, 8 * NRELA)
    gp1 = _edge_precompute(ee1, w1c_t, w2_t).reshape(E1, W)
    gp2 = _edge_precompute(ee2, w1c_t, w2_t).reshape(E2, W)

    src1 = edges[0].astype(jnp.int32).reshape(C1, NSB, SB)
    dst1 = edges[1].astype(jnp.int32).reshape(C1, NSB, SB)
    src2 = nhop_edges[0].astype(jnp.int32).reshape(E2 // K, NSB, SB)
    dst2 = nhop_edges[1].astype(jnp.int32).reshape(E2 // K, NSB, SB)

    p = _sc_aggregate(a_tab, b_tab, gp1, gp2, src1, dst1, src2, dst2)
    return _normalize(p.reshape(NC, NPAD, W))


# speedup vs baseline: 1.0018x; 1.0018x over previous
import functools

import jax
import jax.numpy as jnp
from jax import lax
from jax.experimental import pallas as pl
from jax.experimental.pallas import tpu as pltpu
from jax.experimental.pallas import tpu_sc as plsc

N = 10000
IN_F = 128
OUT_F = 32
NRELA = 16
E1 = 256000
E2 = 64000
E_TOT = E1 + E2

NC = 2
NS = 16
NW = NC * NS
K = 256
NCHUNKS = E_TOT // K
C1 = E1 // K
SB = 128
NSB = K // SB
NG = K // 16
W = 40
NPAD = 10240
RPT = NPAD // NS


def _pack48(w_main, w2_ref):
    scal = lax.dot(w_main, w2_ref, precision=lax.Precision.HIGHEST)
    pad = jnp.zeros((w_main.shape[0], W - OUT_F - 1), jnp.float32)
    return jnp.concatenate([w_main, scal, pad], axis=1)


def _node_body(x_ref, wa_ref, wb_ref, w2_ref, a_ref, b_ref):
    xb = x_ref[...]
    wa48 = _pack48(wa_ref[...], w2_ref[...])
    wb48 = _pack48(wb_ref[...], w2_ref[...])
    a_ref[...] = lax.dot(xb, wa48, precision=lax.Precision.HIGHEST)
    b_ref[...] = lax.dot(xb, wb48, precision=lax.Precision.HIGHEST)


def _node_precompute(x, w1a_t, w1b_t, w2_t):
    blk = 2000
    return pl.pallas_call(
        _node_body,
        grid=(N // blk,),
        in_specs=[
            pl.BlockSpec((blk, IN_F), lambda i: (i, 0)),
            pl.BlockSpec((IN_F, OUT_F), lambda i: (0, 0)),
            pl.BlockSpec((IN_F, OUT_F), lambda i: (0, 0)),
            pl.BlockSpec((OUT_F, 1), lambda i: (0, 0)),
        ],
        out_specs=[
            pl.BlockSpec((blk, W), lambda i: (i, 0)),
            pl.BlockSpec((blk, W), lambda i: (i, 0)),
        ],
        out_shape=[
            jax.ShapeDtypeStruct((N, W), jnp.float32),
            jax.ShapeDtypeStruct((N, W), jnp.float32),
        ],
    )(x, w1a_t, w1b_t, w2_t)


def _edge_body(ee_ref, wc_ref, w2_ref, g_ref):
    w48 = _pack48(wc_ref[...], w2_ref[...])
    pieces = []
    for i in range(8):
        row = [w48]
        if i > 0:
            row.insert(0, jnp.zeros((NRELA, W * i), jnp.float32))
        if i < 7:
            row.append(jnp.zeros((NRELA, W * (7 - i)), jnp.float32))
        pieces.append(jnp.concatenate(row, axis=1))
    wbd = jnp.concatenate(pieces, axis=0)
    g_ref[...] = lax.dot(ee_ref[...], wbd, precision=lax.Precision.HIGHEST)


def _edge_precompute(ee8, w1c_t, w2_t):
    blk = 4000
    nr = ee8.shape[0]
    return pl.pallas_call(
        _edge_body,
        grid=(nr // blk,),
        in_specs=[
            pl.BlockSpec((blk, 8 * NRELA), lambda i: (i, 0)),
            pl.BlockSpec((NRELA, OUT_F), lambda i: (0, 0)),
            pl.BlockSpec((OUT_F, 1), lambda i: (0, 0)),
        ],
        out_specs=pl.BlockSpec((blk, 8 * W), lambda i: (i, 0)),
        out_shape=jax.ShapeDtypeStruct((nr, 8 * W), jnp.float32),
    )(ee8, w1c_t, w2_t)


_BCAST_DNUMS = lax.GatherDimensionNumbers(
    offset_dims=(), collapsed_slice_dims=(0,), start_index_map=(0,))

_MESH = plsc.VectorSubcoreMesh(
    core_axis_name="c", subcore_axis_name="s", num_cores=NC)


_SC_SCRATCH = []
for _b in range(2):
    _SC_SCRATCH += [
        pltpu.VMEM((NSB, SB), jnp.int32),
        pltpu.VMEM((NSB, SB), jnp.int32),
        pltpu.VMEM((NSB, SB), jnp.int32),
        pltpu.VMEM((K, W), jnp.float32),
        pltpu.VMEM((K, W), jnp.float32),
        pltpu.VMEM((K, W), jnp.float32),
        pltpu.VMEM((K, W), jnp.float32),
    ]
_SC_SCRATCH += [pltpu.VMEM_SHARED((NPAD, W), jnp.float32)]
_SC_SCRATCH += [pltpu.SemaphoreType.DMA] * 8


@functools.partial(
    pl.kernel,
    out_type=jax.ShapeDtypeStruct((NC * NPAD, W), jnp.float32),
    mesh=_MESH,
    compiler_params=pltpu.CompilerParams(
        needs_layout_passes=False, use_tc_tiling_on_sc=False),
    scratch_types=_SC_SCRATCH,
)
def _sc_aggregate(a_hbm, b_hbm, gp1_hbm, gp2_hbm,
                  src1_hbm, dst1_hbm, src2_hbm, dst2_hbm, out_hbm,
                  ig_s0, ig_d0, is_s0, ra0, rb0, gv0, wc0,
                  ig_s1, ig_d1, is_s1, ra1, rb1, gv1, wc1,
                  acc, sem_ig0, sem_ig1, sem_is0, sem_is1,
                  sem_g0, sem_g1, sem_sc0, sem_sc1):
    cid = lax.axis_index("c")
    sid = lax.axis_index("s")
    wid = cid * NS + sid
    over = jnp.maximum(wid - 17, 0)
    start = 40 * wid - 2 * over
    nch = jnp.where(wid >= 17, 38, 40)

    bufs = ((ig_s0, ig_d0, is_s0, ra0, rb0, gv0, wc0),
            (ig_s1, ig_d1, is_s1, ra1, rb1, gv1, wc1))
    sem_ig = (sem_ig0, sem_ig1)
    sem_is = (sem_is0, sem_is1)
    sem_g = (sem_g0, sem_g1)
    sem_sc = (sem_sc0, sem_sc1)

    def issue_idx(c, dst_s, dst_d, sem):
        @pl.when(c < C1)
        def _():
            pltpu.async_copy(src1_hbm.at[c], dst_s, sem)
            if dst_d is not None:
                pltpu.async_copy(dst1_hbm.at[c], dst_d, sem)

        @pl.when(c >= C1)
        def _():
            pltpu.async_copy(src2_hbm.at[c - C1], dst_s, sem)
            if dst_d is not None:
                pltpu.async_copy(dst2_hbm.at[c - C1], dst_d, sem)

    def issue_gathers(c, b, sem):
        ig_si, ig_di, _, ra, rb, gv, _ = bufs[b]

        @pl.when(c < C1)
        def _():
            pltpu.async_copy(gp1_hbm.at[pl.ds(c * K, K)], gv, sem)

        @pl.when(c >= C1)
        def _():
            pltpu.async_copy(gp2_hbm.at[pl.ds((c - C1) * K, K)], gv, sem)

        for s in range(NSB):
            pltpu.async_copy(a_hbm.at[ig_si.at[s]],
                             ra.at[pl.ds(s * SB, SB)], sem)
            pltpu.async_copy(b_hbm.at[ig_di.at[s]],
                             rb.at[pl.ds(s * SB, SB)], sem)

    def drain_gathers(b):
        _, _, _, ra, rb, gv, _ = bufs[b]
        sem = sem_g[b]
        pltpu.make_async_copy(gp1_hbm.at[pl.ds(0, K)], gv, sem).wait()
        for s in range(NSB):
            pltpu.make_async_copy(a_hbm.at[pl.ds(0, SB)],
                                  ra.at[pl.ds(s * SB, SB)], sem).wait()
            pltpu.make_async_copy(a_hbm.at[pl.ds(0, SB)],
                                  rb.at[pl.ds(s * SB, SB)], sem).wait()

    def drain_idx(b, scatter):
        if scatter:
            dst, sem = bufs[b][2], sem_is[b]
            pltpu.make_async_copy(src1_hbm.at[0], dst, sem).wait()
        else:
            sem = sem_ig[b]
            pltpu.make_async_copy(src1_hbm.at[0], bufs[b][0], sem).wait()
            pltpu.make_async_copy(src1_hbm.at[0], bufs[b][1], sem).wait()

    def issue_scatter(b):
        is_si, wc = bufs[b][2], bufs[b][6]
        for s in range(NSB):
            pltpu.async_copy(wc.at[pl.ds(s * SB, SB)],
                             acc.at[is_si.at[s]], sem_sc[b], add=True)

    def drain_scatter(b):
        wc = bufs[b][6]
        for s in range(NSB):
            pltpu.make_async_copy(gp1_hbm.at[pl.ds(0, SB)],
                                  wc.at[pl.ds(s * SB, SB)],
                                  sem_sc[b]).wait()

    def compute(b):
        _, _, _, ra, rb, gv, wc = bufs[b]

        @pl.loop(0, NG)
        def _group(g):
            gb = g * 16
            rows16 = jnp.full((16,), gb, jnp.int32) + lax.iota(jnp.int32, 16)
            col32 = jnp.full((16,), OUT_F, jnp.int32)
            a2g = plsc.load_gather(ra, [rows16, col32])
            b2g = plsc.load_gather(rb, [rows16, col32])
            g2g = plsc.load_gather(gv, [rows16, col32])
            braw = a2g + b2g + g2g
            w16 = jnp.exp(jnp.maximum(braw, 0.2 * braw))
            plsc.store_scatter(wc, [rows16, col32], w16)
            for k in range(16):
                wb = lax.gather(
                    w16, jnp.full((16, 1), k, jnp.int32),
                    _BCAST_DNUMS, (1,),
                    mode=lax.GatherScatterMode.PROMISE_IN_BOUNDS)
                r = gb + k
                c0 = (ra[r, pl.ds(0, 16)] + rb[r, pl.ds(0, 16)]
                      + gv[r, pl.ds(0, 16)])
                c1 = (ra[r, pl.ds(16, 16)] + rb[r, pl.ds(16, 16)]
                      + gv[r, pl.ds(16, 16)])
                wc[r, pl.ds(0, 16)] = wb * c0
                wc[r, pl.ds(16, 16)] = wb * c1

    zero = jnp.zeros((16,), jnp.float32)

    @pl.loop(0, K)
    def _zero_row(r):
        for c0 in (0, 16, W - 16):
            wc0[r, pl.ds(c0, 16)] = zero
            wc1[r, pl.ds(c0, 16)] = zero

    for t in range(RPT // 128):
        pltpu.sync_copy(wc0.at[pl.ds(0, 128)],
                        acc.at[pl.ds(sid * RPT + t * 128, 128)])
    plsc.subcore_barrier()

    @pl.when(start < C1)
    def _():
        pltpu.sync_copy(src1_hbm.at[start], ig_s0)
        pltpu.sync_copy(dst1_hbm.at[start], ig_d0)

    @pl.when(start >= C1)
    def _():
        pltpu.sync_copy(src2_hbm.at[start - C1], ig_s0)
        pltpu.sync_copy(dst2_hbm.at[start - C1], ig_d0)

    issue_gathers(start, 0, sem_g[0])
    issue_idx(start + 1, ig_s1, ig_d1, sem_ig[1])

    def part(l, b):
        o = 1 - b
        c = start + l

        @pl.when(l > 0)
        def _():
            drain_scatter(o)

        drain_gathers(b)
        issue_idx(c, bufs[b][2], None, sem_is[b])

        @pl.when(l + 2 < nch)
        def _():
            issue_idx(c + 2, bufs[b][0], bufs[b][1], sem_ig[b])

        compute(b)

        @pl.when(l + 1 < nch)
        def _():
            drain_idx(o, scatter=False)
            issue_gathers(c + 1, o, sem_g[o])

        drain_idx(b, scatter=True)
        issue_scatter(b)

    @pl.loop(0, 20)
    def _pair(p):
        @pl.when(2 * p < nch)
        def _():
            part(2 * p, 0)
            part(2 * p + 1, 1)

    drain_scatter(1)

    plsc.subcore_barrier()
    pltpu.sync_copy(acc.at[pl.ds(sid * RPT, RPT)],
                    out_hbm.at[pl.ds(cid * NPAD + sid * RPT, RPT)])


def _norm_body(p_ref, o_ref):
    s = p_ref[0] + p_ref[1]
    num = s[:, 0:OUT_F]
    den = s[:, OUT_F:OUT_F + 1]
    o_ref[...] = num / (den + 1e-12)


def _normalize(p):
    blk = 1000
    return pl.pallas_call(
        _norm_body,
        grid=(N // blk,),
        in_specs=[pl.BlockSpec((2, blk, W), lambda i: (0, i, 0))],
        out_specs=pl.BlockSpec((blk, OUT_F), lambda i: (i, 0)),
        out_shape=jax.ShapeDtypeStruct((N, OUT_F), jnp.float32),
    )(p)


def kernel(x, edges, edge_embed, nhop_edges, nhop_edge_embed, w1, w2):
    x = x.astype(jnp.float32)

    w1a_t = w1[:, :IN_F].T
    w1b_t = w1[:, IN_F:2 * IN_F].T
    w1c_t = w1[:, 2 * IN_F:].T
    w2_t = w2.T

    a_tab, b_tab = _node_precompute(x, w1a_t, w1b_t, w2_t)
    ee1 = edge_embed.astype(jnp.float32).reshape(E1 // 8, 8 * NRELA)
    ee2 = nhop_edge_embed.astype(jnp.float32).reshape(E2 // 8, 8 * NRELA)
    gp1 = _edge_precompute(ee1, w1c_t, w2_t).reshape(E1, W)
    gp2 = _edge_precompute(ee2, w1c_t, w2_t).reshape(E2, W)

    src1 = edges[0].astype(jnp.int32).reshape(C1, NSB, SB)
    dst1 = edges[1].astype(jnp.int32).reshape(C1, NSB, SB)
    src2 = nhop_edges[0].astype(jnp.int32).reshape(E2 // K, NSB, SB)
    dst2 = nhop_edges[1].astype(jnp.int32).reshape(E2 // K, NSB, SB)

    p = _sc_aggregate(a_tab, b_tab, gp1, gp2, src1, dst1, src2, dst2)
    return _normalize(p.reshape(NC, NPAD, W))
